# Initial kernel scaffold; baseline (speedup 1.0000x reference)
import functools

import jax
import jax.numpy as jnp
from jax import lax
from jax.experimental import pallas as pl
from jax.experimental.pallas import tpu as pltpu
from jax.experimental.pallas import tpu_sc as plsc

_N = 10000
_E = 320000
_D = 128
_NC = 2
_NS = 16
_NW = _NC * _NS
_EW = _E // _NW
_CH = 80
_NCH = _EW // _CH
_NP = 10240
_RPT = _NP // _NS

_mesh = plsc.VectorSubcoreMesh(core_axis_name="c", subcore_axis_name="s")


@functools.partial(
    pl.kernel,
    out_type=jax.ShapeDtypeStruct((_NC * _NP,), jnp.float32),
    mesh=_mesh,
    scratch_types=[
        pltpu.VMEM((_NCH, _CH), jnp.int32),
        pltpu.VMEM((_NCH, _CH), jnp.int32),
        pltpu.VMEM((_NCH, _CH), jnp.float32),
        pltpu.VMEM((_RPT,), jnp.float32),
        pltpu.VMEM_SHARED((_NP,), jnp.float32),
    ],
)
def _deg_kernel(src_hbm, dst_hbm, w_hbm, deg_out, src_v, dst_v, w_v, zb_v,
                deg_sh):
    cid = lax.axis_index("c")
    sid = lax.axis_index("s")
    wid = cid * _NS + sid
    pltpu.sync_copy(src_hbm.at[wid], src_v)
    pltpu.sync_copy(dst_hbm.at[wid], dst_v)
    pltpu.sync_copy(w_hbm.at[wid], w_v)

    def _zero(i, carry):
        zb_v[pl.ds(i * 16, 16)] = jnp.zeros((16,), jnp.float32)
        return carry

    lax.fori_loop(0, _RPT // 16, _zero, None)
    pltpu.sync_copy(zb_v, deg_sh.at[pl.ds(sid * _RPT, _RPT)])
    plsc.subcore_barrier()

    def _chunk(c, carry):
        for j in range(_CH // 16):
            sl = pl.ds(j * 16, 16)
            s = src_v[c, sl]
            d = dst_v[c, sl]
            wv = w_v[c, sl]
            w_v[c, sl] = jnp.where(s == d, 0.0, wv)
        pltpu.sync_copy(w_v.at[c], deg_sh.at[src_v.at[c]], add=True)
        return carry

    lax.fori_loop(0, _NCH, _chunk, None)
    plsc.subcore_barrier()
    pltpu.sync_copy(deg_sh.at[pl.ds(sid * _RPT, _RPT)],
                    deg_out.at[pl.ds(cid * _NP + sid * _RPT, _RPT)])


def _dis_body(deg_ref, dis_ref):
    deg = deg_ref[0:1, :] + deg_ref[1:2, :]
    safe = jnp.where(deg > 0, deg, 1.0)
    dis_ref[...] = jnp.where(deg > 0, lax.rsqrt(safe), 0.0)


_dis_call = pl.pallas_call(
    _dis_body,
    out_shape=jax.ShapeDtypeStruct((1, _NP), jnp.float32),
)


@functools.partial(
    pl.kernel,
    out_type=jax.ShapeDtypeStruct((_NC * _NP, _D), jnp.float32),
    mesh=_mesh,
    scratch_types=[
        pltpu.VMEM((_NCH, _CH), jnp.int32),
        pltpu.VMEM((_NCH, _CH), jnp.int32),
        pltpu.VMEM((_NCH, _CH), jnp.float32),
        pltpu.VMEM((_NP,), jnp.float32),
        pltpu.VMEM((_CH,), jnp.float32),
        pltpu.VMEM((_CH, _D), jnp.float32),
        pltpu.VMEM_SHARED((_NP, _D), jnp.float32),
    ],
)
def _msg_kernel(feat_hbm, src_hbm, dst_hbm, w_hbm, dis_hbm, out_hbm,
                src_v, dst_v, w_v, dis_v, norm_v, rows_v, acc_sh):
    cid = lax.axis_index("c")
    sid = lax.axis_index("s")
    wid = cid * _NS + sid
    pltpu.sync_copy(src_hbm.at[wid], src_v)
    pltpu.sync_copy(dst_hbm.at[wid], dst_v)
    pltpu.sync_copy(w_hbm.at[wid], w_v)
    pltpu.sync_copy(dis_hbm, dis_v)

    def _zero_rows(i, carry):
        for j in range(_D // 16):
            rows_v[i, pl.ds(j * 16, 16)] = jnp.zeros((16,), jnp.float32)
        return carry

    lax.fori_loop(0, _CH, _zero_rows, None)
    for k in range(_RPT // _CH):
        pltpu.sync_copy(rows_v,
                        acc_sh.at[pl.ds(sid * _RPT + k * _CH, _CH)])
    plsc.subcore_barrier()

    def _chunk(c, carry):
        for j in range(_CH // 16):
            sl = pl.ds(j * 16, 16)
            s = src_v[c, sl]
            d = dst_v[c, sl]
            wv = w_v[c, sl]
            wm = jnp.where(s == d, 0.0, wv)
            nv = plsc.load_gather(dis_v, [s]) * wm * plsc.load_gather(dis_v, [d])
            norm_v[sl] = -nv
        pltpu.sync_copy(feat_hbm.at[src_v.at[c]], rows_v)

        def _scale(k, carry2):
            nval = norm_v[k]
            for j in range(_D // 16):
                sl = pl.ds(j * 16, 16)
                rows_v[k, sl] = rows_v[k, sl] * nval
            return carry2

        lax.fori_loop(0, _CH, _scale, None)
        pltpu.sync_copy(rows_v, acc_sh.at[dst_v.at[c]], add=True)
        return carry

    lax.fori_loop(0, _NCH, _chunk, None)
    plsc.subcore_barrier()
    pltpu.sync_copy(acc_sh.at[pl.ds(sid * _RPT, _RPT)],
                    out_hbm.at[pl.ds(cid * _NP + sid * _RPT, _RPT)])


def _mm_body(relu, x_ref, p_ref, w0_ref, w1_ref, b_ref, o_ref):
    t0 = jnp.dot(x_ref[...], w0_ref[...], preferred_element_type=jnp.float32)
    p = p_ref[0] + p_ref[1]
    t1 = jnp.dot(p, w1_ref[...], preferred_element_type=jnp.float32)
    r = t0 + t1 + b_ref[...]
    o_ref[...] = jnp.maximum(r, 0.0) if relu else r


def _mm(v, parts, w0, w1, b, relu):
    bm = 1024
    return pl.pallas_call(
        functools.partial(_mm_body, relu),
        grid=(_NP // bm,),
        in_specs=[
            pl.BlockSpec((bm, _D), lambda i: (i, 0)),
            pl.BlockSpec((_NC, bm, _D), lambda i: (0, i, 0)),
            pl.BlockSpec((_D, _D), lambda i: (0, 0)),
            pl.BlockSpec((_D, _D), lambda i: (0, 0)),
            pl.BlockSpec((1, _D), lambda i: (0, 0)),
        ],
        out_specs=pl.BlockSpec((bm, _D), lambda i: (i, 0)),
        out_shape=jax.ShapeDtypeStruct((_NP, _D), jnp.float32),
    )(v, parts, w0, w1, b)


def kernel(x, edge_index, edge_weight, W1_0, W1_1, b1, W2_0, W2_1, b2):
    src = edge_index[0].reshape(_NW, _NCH, _CH)
    dst = edge_index[1].reshape(_NW, _NCH, _CH)
    w = edge_weight.reshape(_NW, _NCH, _CH)
    x_pad = jnp.pad(x, ((0, _NP - _N), (0, 0)))

    deg_parts = _deg_kernel(src, dst, w)
    dis = _dis_call(deg_parts.reshape(_NC, _NP)).reshape(_NP)

    p1 = _msg_kernel(x_pad, src, dst, w, dis)
    h = _mm(x_pad, p1.reshape(_NC, _NP, _D), W1_0, W1_1,
            b1.reshape(1, _D), relu=True)
    p2 = _msg_kernel(h, src, dst, w, dis)
    out = _mm(h, p2.reshape(_NC, _NP, _D), W2_0, W2_1,
              b2.reshape(1, _D), relu=False)
    return out[:_N]


# speedup vs baseline: 6.9716x; 6.9716x over previous
import functools

import jax
import jax.numpy as jnp
from jax import lax
from jax.experimental import pallas as pl
from jax.experimental.pallas import tpu as pltpu
from jax.experimental.pallas import tpu_sc as plsc

_N = 10000
_E = 320000
_D = 128
_NC = 2
_NS = 16
_NW = _NC * _NS
_EW = _E // _NW
_CH = 80
_NCH = _EW // _CH
_NP = 10240
_RPT = _NP // _NS

_mesh = plsc.VectorSubcoreMesh(core_axis_name="c", subcore_axis_name="s")
_sc_params = pltpu.CompilerParams(needs_layout_passes=False)


@functools.partial(
    pl.kernel,
    out_type=jax.ShapeDtypeStruct((_NC * _NP,), jnp.float32),
    mesh=_mesh,
    compiler_params=_sc_params,
    scratch_types=[
        pltpu.VMEM((_CH,), jnp.int32),
        pltpu.VMEM((_CH,), jnp.int32),
        pltpu.VMEM((_CH,), jnp.float32),
        pltpu.VMEM((_RPT,), jnp.float32),
        pltpu.VMEM_SHARED((_NP,), jnp.float32),
    ],
)
def _deg_kernel(src_hbm, dst_hbm, w_hbm, deg_out, sidx_v, didx_v, w_v, zb_v,
                deg_sh):
    cid = lax.axis_index("c")
    sid = lax.axis_index("s")
    wid = cid * _NS + sid

    def _zero(i, carry):
        zb_v[pl.ds(i * 16, 16)] = jnp.zeros((16,), jnp.float32)
        return carry

    lax.fori_loop(0, _RPT // 16, _zero, None)
    pltpu.sync_copy(zb_v, deg_sh.at[pl.ds(sid * _RPT, _RPT)])
    plsc.subcore_barrier()

    def _chunk(c, carry):
        pltpu.sync_copy(src_hbm.at[wid, c], sidx_v)
        pltpu.sync_copy(dst_hbm.at[wid, c], didx_v)
        pltpu.sync_copy(w_hbm.at[wid, c], w_v)
        for j in range(_CH // 16):
            sl = pl.ds(j * 16, 16)
            s = sidx_v[sl]
            d = didx_v[sl]
            wv = w_v[sl]
            w_v[sl] = jnp.where(s == d, 0.0, wv)
        pltpu.sync_copy(w_v, deg_sh.at[sidx_v], add=True)
        return carry

    lax.fori_loop(0, _NCH, _chunk, None)
    plsc.subcore_barrier()
    pltpu.sync_copy(deg_sh.at[pl.ds(sid * _RPT, _RPT)],
                    deg_out.at[pl.ds(cid * _NP + sid * _RPT, _RPT)])


def _dis_body(deg_ref, dis_ref):
    deg = deg_ref[0:1, :] + deg_ref[1:2, :]
    safe = jnp.where(deg > 0, deg, 1.0)
    dis_ref[...] = jnp.where(deg > 0, lax.rsqrt(safe), 0.0)


_dis_call = pl.pallas_call(
    _dis_body,
    out_shape=jax.ShapeDtypeStruct((1, _NP), jnp.float32),
)


@functools.partial(
    pl.kernel,
    out_type=jax.ShapeDtypeStruct((_NC * _NP, _D), jnp.float32),
    mesh=_mesh,
    compiler_params=_sc_params,
    scratch_types=[
        pltpu.VMEM((_CH,), jnp.int32),
        pltpu.VMEM((_CH,), jnp.int32),
        pltpu.VMEM((_CH,), jnp.float32),
        pltpu.VMEM((_NP,), jnp.float32),
        pltpu.VMEM((_CH,), jnp.float32),
        pltpu.VMEM((_CH, _D), jnp.float32),
        pltpu.VMEM_SHARED((_NP, _D), jnp.float32),
    ],
)
def _msg_kernel(feat_hbm, src_hbm, dst_hbm, w_hbm, dis_hbm, out_hbm,
                sidx_v, didx_v, w_v, dis_v, norm_v, rows_v, acc_sh):
    cid = lax.axis_index("c")
    sid = lax.axis_index("s")
    wid = cid * _NS + sid
    pltpu.sync_copy(dis_hbm, dis_v)

    def _zero_rows(i, carry):
        for j in range(_D // 16):
            rows_v[i, pl.ds(j * 16, 16)] = jnp.zeros((16,), jnp.float32)
        return carry

    lax.fori_loop(0, _CH, _zero_rows, None)
    for k in range(_RPT // _CH):
        pltpu.sync_copy(rows_v,
                        acc_sh.at[pl.ds(sid * _RPT + k * _CH, _CH)])
    plsc.subcore_barrier()

    def _chunk(c, carry):
        pltpu.sync_copy(src_hbm.at[wid, c], sidx_v)
        pltpu.sync_copy(dst_hbm.at[wid, c], didx_v)
        pltpu.sync_copy(w_hbm.at[wid, c], w_v)
        pltpu.sync_copy(feat_hbm.at[sidx_v], rows_v)
        for j in range(_CH // 16):
            sl = pl.ds(j * 16, 16)
            s = sidx_v[sl]
            d = didx_v[sl]
            wv = w_v[sl]
            wm = jnp.where(s == d, 0.0, wv)
            nv = plsc.load_gather(dis_v, [s]) * wm * plsc.load_gather(dis_v, [d])
            norm_v[sl] = -nv

        def _scale(k, carry2):
            bidx = jnp.full((16,), k, jnp.int32)
            nval = plsc.load_gather(norm_v, [bidx])
            for j in range(_D // 16):
                sl = pl.ds(j * 16, 16)
                rows_v[k, sl] = rows_v[k, sl] * nval
            return carry2

        lax.fori_loop(0, _CH, _scale, None)
        pltpu.sync_copy(rows_v, acc_sh.at[didx_v], add=True)
        return carry

    lax.fori_loop(0, _NCH, _chunk, None)
    plsc.subcore_barrier()
    pltpu.sync_copy(acc_sh.at[pl.ds(sid * _RPT, _RPT)],
                    out_hbm.at[pl.ds(cid * _NP + sid * _RPT, _RPT)])


def _mm_body(relu, x_ref, p_ref, w0_ref, w1_ref, b_ref, o_ref):
    t0 = jnp.dot(x_ref[...], w0_ref[...], preferred_element_type=jnp.float32)
    p = p_ref[0] + p_ref[1]
    t1 = jnp.dot(p, w1_ref[...], preferred_element_type=jnp.float32)
    r = t0 + t1 + b_ref[...]
    o_ref[...] = jnp.maximum(r, 0.0) if relu else r


def _mm(v, parts, w0, w1, b, relu):
    bm = 1024
    return pl.pallas_call(
        functools.partial(_mm_body, relu),
        grid=(_NP // bm,),
        in_specs=[
            pl.BlockSpec((bm, _D), lambda i: (i, 0)),
            pl.BlockSpec((_NC, bm, _D), lambda i: (0, i, 0)),
            pl.BlockSpec((_D, _D), lambda i: (0, 0)),
            pl.BlockSpec((_D, _D), lambda i: (0, 0)),
            pl.BlockSpec((1, _D), lambda i: (0, 0)),
        ],
        out_specs=pl.BlockSpec((bm, _D), lambda i: (i, 0)),
        out_shape=jax.ShapeDtypeStruct((_NP, _D), jnp.float32),
    )(v, parts, w0, w1, b)


def kernel(x, edge_index, edge_weight, W1_0, W1_1, b1, W2_0, W2_1, b2):
    src = edge_index[0].reshape(_NW, _NCH, _CH)
    dst = edge_index[1].reshape(_NW, _NCH, _CH)
    w = edge_weight.reshape(_NW, _NCH, _CH)
    x_pad = jnp.pad(x, ((0, _NP - _N), (0, 0)))

    deg_parts = _deg_kernel(src, dst, w)
    dis = _dis_call(deg_parts.reshape(_NC, _NP)).reshape(_NP)

    p1 = _msg_kernel(x_pad, src, dst, w, dis)
    h = _mm(x_pad, p1.reshape(_NC, _NP, _D), W1_0, W1_1,
            b1.reshape(1, _D), relu=True)
    p2 = _msg_kernel(h, src, dst, w, dis)
    out = _mm(h, p2.reshape(_NC, _NP, _D), W2_0, W2_1,
              b2.reshape(1, _D), relu=False)
    return out[:_N]


# speedup vs baseline: 16.1562x; 2.3174x over previous
import functools

import jax
import jax.numpy as jnp
from jax import lax
from jax.experimental import pallas as pl
from jax.experimental.pallas import tpu as pltpu
from jax.experimental.pallas import tpu_sc as plsc

_N = 10000
_E = 320000
_D = 128
_NC = 2
_NS = 16
_NW = _NC * _NS
_CH = 128
_NCH = 80
_EW = _NCH * _CH
_EP = _NW * _EW
_NP = 10240
_RPT = _NP // _NS
_BLK = 8

_mesh = plsc.VectorSubcoreMesh(core_axis_name="c", subcore_axis_name="s")
_sc_params = pltpu.CompilerParams(needs_layout_passes=False)


def _f32(x):
    return plsc.bitcast(x, jnp.float32)


@functools.partial(
    pl.kernel,
    out_type=jax.ShapeDtypeStruct((_NC * _NP,), jnp.float32),
    mesh=_mesh,
    compiler_params=_sc_params,
    scratch_types=[
        pltpu.VMEM((_BLK * 3, _CH), jnp.int32),
        pltpu.VMEM((_BLK * 3, _CH), jnp.int32),
        pltpu.VMEM((_BLK, _CH), jnp.float32),
        pltpu.VMEM((_BLK, _CH), jnp.float32),
        pltpu.VMEM((_RPT,), jnp.float32),
        pltpu.VMEM_SHARED((_NP,), jnp.float32),
        pltpu.SemaphoreType.DMA,
        pltpu.SemaphoreType.DMA,
        pltpu.SemaphoreType.DMA,
        pltpu.SemaphoreType.DMA,
    ],
)
def _deg_kernel(meta_hbm, deg_out, mb0_v, mb1_v, wb0_v, wb1_v, zb_v, deg_sh,
                dm0, dm1, fs0, fs1):
    cid = lax.axis_index("c")
    sid = lax.axis_index("s")
    wid = cid * _NS + sid
    mbs = (mb0_v, mb1_v)
    wbs = (wb0_v, wb1_v)
    dms = (dm0, dm1)
    fss = (fs0, fs1)

    def _zero(i, carry):
        zb_v[pl.ds(i * 16, 16)] = jnp.zeros((16,), jnp.float32)
        return carry

    lax.fori_loop(0, _RPT // 16, _zero, None)
    pltpu.sync_copy(zb_v, deg_sh.at[pl.ds(sid * _RPT, _RPT)])
    plsc.subcore_barrier()

    nblk = _NCH // _BLK

    def _meta_copy(b, k):
        return pltpu.make_async_copy(
            meta_hbm.at[wid, pl.ds(k * _BLK * 3, _BLK * 3)], mbs[b], dms[b])

    def _fire(b, j):
        return pltpu.async_copy(
            wbs[b].at[j], deg_sh.at[mbs[b].at[j * 3]], fss[b], add=True)

    def _drain(b, j):
        pltpu.make_async_copy(
            wbs[b].at[j], deg_sh.at[mbs[b].at[j * 3]], fss[b]).wait()

    _meta_copy(0, 0).start()

    def _outer(kk, carry):
        for b in range(2):
            k = kk * 2 + b
            _meta_copy(b, k).wait()
            for j in range(_BLK):
                for g in range(_CH // 16):
                    sl = pl.ds(g * 16, 16)
                    s = mbs[b][j * 3, sl]
                    d = mbs[b][j * 3 + 1, sl]
                    wv = _f32(mbs[b][j * 3 + 2, sl])
                    wbs[b][j, sl] = jnp.where(s == d, 0.0, wv)

            @pl.when(k >= 1)
            def _():
                for j in range(_BLK):
                    _drain(1 - b, j)

            @pl.when(k + 1 < nblk)
            def _():
                _meta_copy(1 - b, k + 1).start()

            for j in range(_BLK):
                _fire(b, j)
        return carry

    lax.fori_loop(0, nblk // 2, _outer, None)
    for j in range(_BLK):
        _drain(1, j)
    plsc.subcore_barrier()
    pltpu.sync_copy(deg_sh.at[pl.ds(sid * _RPT, _RPT)],
                    deg_out.at[pl.ds(cid * _NP + sid * _RPT, _RPT)])


def _dis_body(deg_ref, dis_ref):
    deg = deg_ref[0:1, :] + deg_ref[1:2, :]
    safe = jnp.where(deg > 0, deg, 1.0)
    dis_ref[...] = jnp.where(deg > 0, lax.rsqrt(safe), 0.0)


_dis_call = pl.pallas_call(
    _dis_body,
    out_shape=jax.ShapeDtypeStruct((1, _NP), jnp.float32),
)


@functools.partial(
    pl.kernel,
    out_type=jax.ShapeDtypeStruct((_NC * _NP, _D), jnp.float32),
    mesh=_mesh,
    compiler_params=_sc_params,
    scratch_types=[
        pltpu.VMEM((3, _CH), jnp.int32),
        pltpu.VMEM((3, _CH), jnp.int32),
        pltpu.VMEM((_NP,), jnp.float32),
        pltpu.VMEM((_CH,), jnp.float32),
        pltpu.VMEM((_CH, _D), jnp.float32),
        pltpu.VMEM((_CH, _D), jnp.float32),
        pltpu.VMEM_SHARED((_NP, _D), jnp.float32),
        pltpu.SemaphoreType.DMA,
        pltpu.SemaphoreType.DMA,
        pltpu.SemaphoreType.DMA,
        pltpu.SemaphoreType.DMA,
        pltpu.SemaphoreType.DMA,
        pltpu.SemaphoreType.DMA,
    ],
)
def _msg_kernel(feat_hbm, meta_hbm, dis_hbm, out_hbm,
                mb0_v, mb1_v, dis_v, norm_v, rows0_v, rows1_v, acc_sh,
                mm0, mm1, gg0, gg1, ss0, ss1):
    cid = lax.axis_index("c")
    sid = lax.axis_index("s")
    wid = cid * _NS + sid
    mbs = (mb0_v, mb1_v)
    rows = (rows0_v, rows1_v)
    mms = (mm0, mm1)
    ggs = (gg0, gg1)
    sss = (ss0, ss1)

    pltpu.sync_copy(dis_hbm, dis_v)

    def _zero_rows(i, carry):
        for j in range(_D // 16):
            rows0_v[i, pl.ds(j * 16, 16)] = jnp.zeros((16,), jnp.float32)
        return carry

    lax.fori_loop(0, _CH, _zero_rows, None)
    for k in range(_RPT // _CH):
        pltpu.sync_copy(rows0_v,
                        acc_sh.at[pl.ds(sid * _RPT + k * _CH, _CH)])
    plsc.subcore_barrier()

    def _meta_copy(b, c):
        return pltpu.make_async_copy(meta_hbm.at[wid, c], mbs[b], mms[b])

    def _gather(b):
        return pltpu.make_async_copy(
            feat_hbm.at[mbs[b].at[0]], rows[b], ggs[b])

    def _scatter_start(b):
        pltpu.async_copy(rows[b], acc_sh.at[mbs[b].at[1]], sss[b],
                         add=True)

    def _scatter_wait(b):
        pltpu.make_async_copy(
            rows[b], acc_sh.at[mbs[b].at[1]], sss[b]).wait()

    def _norm(b):
        for g in range(_CH // 16):
            sl = pl.ds(g * 16, 16)
            s = mbs[b][0, sl]
            d = mbs[b][1, sl]
            wv = _f32(mbs[b][2, sl])
            wm = jnp.where(s == d, 0.0, wv)
            nv = plsc.load_gather(dis_v, [s]) * wm * plsc.load_gather(dis_v, [d])
            norm_v[sl] = -nv

    def _scale(b):
        def _srow(k, carry2):
            bidx = jnp.full((16,), k, jnp.int32)
            nval = plsc.load_gather(norm_v, [bidx])
            for j in range(_D // 16):
                sl = pl.ds(j * 16, 16)
                rows[b][k, sl] = rows[b][k, sl] * nval
            return carry2

        lax.fori_loop(0, _CH, _srow, None, unroll=2)

    _meta_copy(0, 0).start()

    def _outer(cc, carry):
        for b in range(2):
            c = cc * 2 + b
            _meta_copy(b, c).wait()
            _gather(b).start()
            _norm(b)

            @pl.when(c >= 1)
            def _():
                _scatter_wait(1 - b)

            @pl.when(c + 1 < _NCH)
            def _():
                _meta_copy(1 - b, c + 1).start()

            _gather(b).wait()
            _scale(b)
            _scatter_start(b)
        return carry

    lax.fori_loop(0, _NCH // 2, _outer, None)
    _scatter_wait(1)
    plsc.subcore_barrier()
    pltpu.sync_copy(acc_sh.at[pl.ds(sid * _RPT, _RPT)],
                    out_hbm.at[pl.ds(cid * _NP + sid * _RPT, _RPT)])


def _mm_body(relu, x_ref, p_ref, w0_ref, w1_ref, b_ref, o_ref):
    t0 = jnp.dot(x_ref[...], w0_ref[...], preferred_element_type=jnp.float32)
    p = p_ref[0] + p_ref[1]
    t1 = jnp.dot(p, w1_ref[...], preferred_element_type=jnp.float32)
    r = t0 + t1 + b_ref[...]
    o_ref[...] = jnp.maximum(r, 0.0) if relu else r


def _mm(v, parts, w0, w1, b, relu):
    bm = 1024
    return pl.pallas_call(
        functools.partial(_mm_body, relu),
        grid=(_NP // bm,),
        in_specs=[
            pl.BlockSpec((bm, _D), lambda i: (i, 0)),
            pl.BlockSpec((_NC, bm, _D), lambda i: (0, i, 0)),
            pl.BlockSpec((_D, _D), lambda i: (0, 0)),
            pl.BlockSpec((_D, _D), lambda i: (0, 0)),
            pl.BlockSpec((1, _D), lambda i: (0, 0)),
        ],
        out_specs=pl.BlockSpec((bm, _D), lambda i: (i, 0)),
        out_shape=jax.ShapeDtypeStruct((_NP, _D), jnp.float32),
    )(v, parts, w0, w1, b)


def kernel(x, edge_index, edge_weight, W1_0, W1_1, b1, W2_0, W2_1, b2):
    npad = _EP - _E
    pad_ids = _N + (jnp.arange(npad, dtype=jnp.int32) % (_NP - _N))
    src = jnp.concatenate([edge_index[0], pad_ids]).reshape(_NW, _NCH, _CH)
    dst = jnp.concatenate([edge_index[1], pad_ids]).reshape(_NW, _NCH, _CH)
    wi = lax.bitcast_convert_type(edge_weight, jnp.int32)
    w = jnp.concatenate([wi, jnp.zeros((npad,), jnp.int32)]
                        ).reshape(_NW, _NCH, _CH)
    meta = jnp.stack([src, dst, w], axis=2)
    meta2 = meta.reshape(_NW, _NCH * 3, _CH)
    x_pad = jnp.pad(x, ((0, _NP - _N), (0, 0)))

    deg_parts = _deg_kernel(meta2)
    dis = _dis_call(deg_parts.reshape(_NC, _NP)).reshape(_NP)

    p1 = _msg_kernel(x_pad, meta, dis)
    h = _mm(x_pad, p1.reshape(_NC, _NP, _D), W1_0, W1_1,
            b1.reshape(1, _D), relu=True)
    p2 = _msg_kernel(h, meta, dis)
    out = _mm(h, p2.reshape(_NC, _NP, _D), W2_0, W2_1,
              b2.reshape(1, _D), relu=False)
    return out[:_N]


# speedup vs baseline: 26.2161x; 1.6227x over previous
import functools

import jax
import jax.numpy as jnp
from jax import lax
from jax.experimental import pallas as pl
from jax.experimental.pallas import tpu as pltpu
from jax.experimental.pallas import tpu_sc as plsc

_N = 10000
_E = 320000
_D = 128
_NC = 2
_NS = 16
_NW = _NC * _NS
_NP = 10240
_RPT = _NP // _NS

_CH = 96
_NCH = 105
_EW = _NCH * _CH
_EP = _NW * _EW

_CHD = 128
_NCHD = 80
_EWD = _NCHD * _CHD
_EPD = _NW * _EWD
_BLK = 8

_mesh = plsc.VectorSubcoreMesh(core_axis_name="c", subcore_axis_name="s")
_sc_params = pltpu.CompilerParams(needs_layout_passes=False)


def _f32(x):
    return plsc.bitcast(x, jnp.float32)


@functools.partial(
    pl.kernel,
    out_type=jax.ShapeDtypeStruct((_NC * _NP,), jnp.float32),
    mesh=_mesh,
    compiler_params=_sc_params,
    scratch_types=[
        pltpu.VMEM((_BLK * 3, _CHD), jnp.int32),
        pltpu.VMEM((_BLK * 3, _CHD), jnp.int32),
        pltpu.VMEM((_BLK, _CHD), jnp.float32),
        pltpu.VMEM((_BLK, _CHD), jnp.float32),
        pltpu.VMEM((_RPT,), jnp.float32),
        pltpu.VMEM_SHARED((_NP,), jnp.float32),
        pltpu.SemaphoreType.DMA,
        pltpu.SemaphoreType.DMA,
        pltpu.SemaphoreType.DMA,
        pltpu.SemaphoreType.DMA,
    ],
)
def _deg_kernel(meta_hbm, deg_out, mb0_v, mb1_v, wb0_v, wb1_v, zb_v, deg_sh,
                dm0, dm1, fs0, fs1):
    cid = lax.axis_index("c")
    sid = lax.axis_index("s")
    wid = cid * _NS + sid
    mbs = (mb0_v, mb1_v)
    wbs = (wb0_v, wb1_v)
    dms = (dm0, dm1)
    fss = (fs0, fs1)

    def _zero(i, carry):
        zb_v[pl.ds(i * 16, 16)] = jnp.zeros((16,), jnp.float32)
        return carry

    lax.fori_loop(0, _RPT // 16, _zero, None)
    pltpu.sync_copy(zb_v, deg_sh.at[pl.ds(sid * _RPT, _RPT)])
    plsc.subcore_barrier()

    nblk = _NCHD // _BLK

    def _meta_copy(b, k):
        return pltpu.make_async_copy(
            meta_hbm.at[wid, pl.ds(k * _BLK * 3, _BLK * 3)], mbs[b], dms[b])

    def _fire(b, j):
        return pltpu.async_copy(
            wbs[b].at[j], deg_sh.at[mbs[b].at[j * 3]], fss[b], add=True)

    def _drain(b, j):
        pltpu.make_async_copy(
            wbs[b].at[j], deg_sh.at[mbs[b].at[j * 3]], fss[b]).wait()

    _meta_copy(0, 0).start()

    def _outer(kk, carry):
        for b in range(2):
            k = kk * 2 + b
            _meta_copy(b, k).wait()
            for j in range(_BLK):
                for g in range(_CHD // 16):
                    sl = pl.ds(g * 16, 16)
                    s = mbs[b][j * 3, sl]
                    d = mbs[b][j * 3 + 1, sl]
                    wv = _f32(mbs[b][j * 3 + 2, sl])
                    wbs[b][j, sl] = jnp.where(s == d, 0.0, wv)

            @pl.when(k >= 1)
            def _():
                for j in range(_BLK):
                    _drain(1 - b, j)

            @pl.when(k + 1 < nblk)
            def _():
                _meta_copy(1 - b, k + 1).start()

            for j in range(_BLK):
                _fire(b, j)
        return carry

    lax.fori_loop(0, nblk // 2, _outer, None)
    for j in range(_BLK):
        _drain(1, j)
    plsc.subcore_barrier()
    pltpu.sync_copy(deg_sh.at[pl.ds(sid * _RPT, _RPT)],
                    deg_out.at[pl.ds(cid * _NP + sid * _RPT, _RPT)])


def _dis_body(deg_ref, dis_ref):
    deg = deg_ref[0:1, :] + deg_ref[1:2, :]
    safe = jnp.where(deg > 0, deg, 1.0)
    dis_ref[...] = jnp.where(deg > 0, lax.rsqrt(safe), 0.0)


_dis_call = pl.pallas_call(
    _dis_body,
    out_shape=jax.ShapeDtypeStruct((1, _NP), jnp.float32),
)


@functools.partial(
    pl.kernel,
    out_type=jax.ShapeDtypeStruct((_NC * _NP, _D), jnp.float32),
    mesh=_mesh,
    compiler_params=_sc_params,
    scratch_types=[
        [pltpu.VMEM((3, _CH), jnp.int32)] * 3,
        [pltpu.VMEM((_CH,), jnp.int32)] * 3,
        [pltpu.VMEM((_CH, _D), jnp.float32)] * 3,
        pltpu.VMEM((_NP,), jnp.float32),
        pltpu.VMEM((_CH,), jnp.float32),
        pltpu.VMEM_SHARED((_NP, _D), jnp.float32),
        [pltpu.SemaphoreType.DMA] * 3,
        [pltpu.SemaphoreType.DMA] * 3,
        [pltpu.SemaphoreType.DMA] * 3,
        pltpu.SemaphoreType.DMA,
    ],
)
def _msg_kernel(feat_hbm, meta_hbm, dis_hbm, out_hbm,
                mbs, didx, rows, dis_v, norm_v, acc_sh, mms, ggs, sss, ds):
    cid = lax.axis_index("c")
    sid = lax.axis_index("s")
    wid = cid * _NS + sid

    def _meta_copy(b, c):
        return pltpu.make_async_copy(meta_hbm.at[wid, c], mbs[b], mms[b])

    def _gather(b):
        return pltpu.make_async_copy(
            feat_hbm.at[mbs[b].at[0]], rows[b], ggs[b])

    def _scatter_start(b):
        pltpu.async_copy(rows[b], acc_sh.at[didx[b]], sss[b], add=True)

    def _scatter_wait(b):
        pltpu.make_async_copy(rows[b], acc_sh.at[didx[b]], sss[b]).wait()

    def _norm(b):
        for g in range(_CH // 16):
            sl = pl.ds(g * 16, 16)
            s = mbs[b][0, sl]
            d = mbs[b][1, sl]
            didx[b][sl] = d
            wv = _f32(mbs[b][2, sl])
            wm = jnp.where(s == d, 0.0, wv)
            nv = plsc.load_gather(dis_v, [s]) * wm * plsc.load_gather(dis_v, [d])
            norm_v[sl] = -nv

    def _scale(b):
        def _sgrp(g, carry2):
            nv = norm_v[pl.ds(g * 16, 16)]
            for r in range(16):
                bc = jnp.take_along_axis(
                    nv, jnp.full((16,), r, jnp.int32), axis=0)
                k = g * 16 + r
                for j in range(_D // 16):
                    sl = pl.ds(j * 16, 16)
                    rows[b][k, sl] = rows[b][k, sl] * bc
            return carry2

        lax.fori_loop(0, _CH // 16, _sgrp, None)

    _meta_copy(0, 0).start()
    _meta_copy(1, 1).start()
    dis_cp = pltpu.make_async_copy(dis_hbm, dis_v, ds)
    dis_cp.start()

    def _zero_rows(i, carry):
        for j in range(_D // 16):
            rows[2][i, pl.ds(j * 16, 16)] = jnp.zeros((16,), jnp.float32)
        return carry

    lax.fori_loop(0, _CH, _zero_rows, None)
    _meta_copy(0, 0).wait()
    _gather(0).start()
    for k in range(_RPT // _CH):
        pltpu.sync_copy(rows[2],
                        acc_sh.at[pl.ds(sid * _RPT + k * _CH, _CH)])
    _rem = _RPT % _CH
    if _rem:
        pltpu.sync_copy(
            rows[2].at[pl.ds(0, _rem)],
            acc_sh.at[pl.ds(sid * _RPT + (_RPT // _CH) * _CH, _rem)])
    dis_cp.wait()
    plsc.subcore_barrier()

    def _outer(cc, carry):
        for b in range(3):
            c = cc * 3 + b
            b1 = (b + 1) % 3
            b2 = (b + 2) % 3

            @pl.when(c + 1 < _NCH)
            def _():
                _meta_copy(b1, c + 1).wait()

            @pl.when(c + 2 < _NCH)
            def _():
                _meta_copy(b2, c + 2).start()

            @pl.when(c >= 2)
            def _():
                _scatter_wait(b1)

            @pl.when(c + 1 < _NCH)
            def _():
                _gather(b1).start()

            _gather(b).wait()
            _norm(b)
            _scale(b)
            _scatter_start(b)
        return carry

    lax.fori_loop(0, _NCH // 3, _outer, None)
    for s in ((_NCH - 2) % 3, (_NCH - 1) % 3):
        _scatter_wait(s)
    plsc.subcore_barrier()
    pltpu.sync_copy(acc_sh.at[pl.ds(sid * _RPT, _RPT)],
                    out_hbm.at[pl.ds(cid * _NP + sid * _RPT, _RPT)])


def _mm_body(relu, x_ref, p_ref, w0_ref, w1_ref, b_ref, o_ref):
    t0 = jnp.dot(x_ref[...], w0_ref[...], preferred_element_type=jnp.float32)
    p = p_ref[0] + p_ref[1]
    t1 = jnp.dot(p, w1_ref[...], preferred_element_type=jnp.float32)
    r = t0 + t1 + b_ref[...]
    o_ref[...] = jnp.maximum(r, 0.0) if relu else r


def _mm(v, parts, w0, w1, b, relu):
    bm = 1000
    return pl.pallas_call(
        functools.partial(_mm_body, relu),
        grid=(_N // bm,),
        in_specs=[
            pl.BlockSpec((bm, _D), lambda i: (i, 0)),
            pl.BlockSpec((_NC, bm, _D), lambda i: (0, i, 0)),
            pl.BlockSpec((_D, _D), lambda i: (0, 0)),
            pl.BlockSpec((_D, _D), lambda i: (0, 0)),
            pl.BlockSpec((1, _D), lambda i: (0, 0)),
        ],
        out_specs=pl.BlockSpec((bm, _D), lambda i: (i, 0)),
        out_shape=jax.ShapeDtypeStruct((_N, _D), jnp.float32),
    )(v, parts, w0, w1, b)


def _pack_meta(edge_index, edge_weight, ep, nch, ch):
    npad = ep - _E
    pad_ids = jnp.arange(npad, dtype=jnp.int32) % _N
    src = jnp.concatenate([edge_index[0], pad_ids]).reshape(_NW, nch, ch)
    dst = jnp.concatenate([edge_index[1], pad_ids]).reshape(_NW, nch, ch)
    wi = lax.bitcast_convert_type(edge_weight, jnp.int32)
    w = jnp.concatenate([wi, jnp.zeros((npad,), jnp.int32)]
                        ).reshape(_NW, nch, ch)
    return jnp.stack([src, dst, w], axis=2)


def kernel(x, edge_index, edge_weight, W1_0, W1_1, b1, W2_0, W2_1, b2):
    meta = _pack_meta(edge_index, edge_weight, _EP, _NCH, _CH)
    meta_deg = _pack_meta(edge_index, edge_weight, _EPD, _NCHD, _CHD)
    meta_deg = meta_deg.reshape(_NW, _NCHD * 3, _CHD)

    deg_parts = _deg_kernel(meta_deg)
    dis = _dis_call(deg_parts.reshape(_NC, _NP)).reshape(_NP)

    p1 = _msg_kernel(x, meta, dis)
    h = _mm(x, p1.reshape(_NC, _NP, _D), W1_0, W1_1,
            b1.reshape(1, _D), relu=True)
    p2 = _msg_kernel(h, meta, dis)
    return _mm(h, p2.reshape(_NC, _NP, _D), W2_0, W2_1,
               b2.reshape(1, _D), relu=False)
